# baseline (device time: 68213 ns/iter reference)
import functools

import jax
import jax.numpy as jnp
from jax import lax
from jax.experimental import pallas as pl
from jax.experimental.pallas import tpu as pltpu

N_DEV = 4
SCALE = 0.08838834764831843
DH = 128
LOCAL_WINDOW = 128
GLOBAL_K = 32


def _fused_body(
    x_ref, wq_ref, k_ref, v_ref, wo_ref, out_ref,
    p_s, r1, a_s, r2, send_sems, recv_sems,
):
    my = lax.axis_index("i")
    p1 = my ^ 1
    p2 = 3 - my
    partner = [[p1, p2], [p2, p1]]

    sq, d = out_ref.shape
    hc = sq // 2
    cw = d // 2
    hq_local = k_ref.shape[1]

    barrier_sem = pltpu.get_barrier_semaphore()
    for nbr in [p1, p2]:
        pl.semaphore_signal(
            barrier_sem, inc=1,
            device_id=(nbr,), device_id_type=pl.DeviceIdType.MESH,
        )
    pl.semaphore_wait(barrier_sem, 2)

    qm = jnp.dot(
        x_ref[...], wq_ref[...], preferred_element_type=jnp.float32
    ).astype(jnp.bfloat16)

    def bias_band(row0, nrows, col_pieces):
        parts = []
        for c0, w in col_pieces:
            qi = lax.broadcasted_iota(jnp.int32, (nrows, w), 0) + row0
            ki = lax.broadcasted_iota(jnp.int32, (nrows, w), 1) + c0
            m = (
                (jnp.abs(qi - ki) <= LOCAL_WINDOW)
                | (ki < GLOBAL_K)
                | (qi < GLOBAL_K)
            )
            parts.append(jnp.where(m, jnp.float32(0.0), jnp.float32(-1e9)))
        return jnp.concatenate(parts, axis=1) if len(parts) > 1 else parts[0]

    chunk_bands = [
        [(0, 128, [(0, sq)]), (128, 384, [(0, 640)])],
        [(512, 512, [(0, 128), (384, 640)])],
    ]
    chunk_biases = [
        [bias_band(r0, nr, cp) for r0, nr, cp in bands]
        for bands in chunk_bands
    ]

    def compute_chunk(c):
        head_parts = []
        for hh in range(hq_local):
            band_rows = []
            for (r0, nr, cp), bias in zip(chunk_bands[c], chunk_biases[c]):
                qh = qm[r0:r0 + nr, hh * DH:(hh + 1) * DH]
                if len(cp) == 1:
                    c0, w = cp[0]
                    ksub = k_ref[c0:c0 + w, hh, :]
                    vsub = v_ref[c0:c0 + w, hh, :]
                else:
                    ksub = jnp.concatenate(
                        [k_ref[c0:c0 + w, hh, :] for c0, w in cp], axis=0
                    )
                    vsub = jnp.concatenate(
                        [v_ref[c0:c0 + w, hh, :] for c0, w in cp], axis=0
                    )
                s = lax.dot_general(
                    qh, ksub, (((1,), (1,)), ((), ())),
                    preferred_element_type=jnp.float32,
                ) * SCALE + bias
                e = jnp.exp(s)
                den = jnp.sum(e, axis=1, keepdims=True)
                w = (e / den).astype(jnp.bfloat16)
                band_rows.append(
                    jnp.dot(
                        w, vsub, preferred_element_type=jnp.float32
                    ).astype(jnp.bfloat16)
                )
            head_parts.append(
                band_rows[0] if len(band_rows) == 1
                else jnp.concatenate(band_rows, axis=0)
            )
        ctx = jnp.concatenate(head_parts, axis=1)
        p_s[c * hc:(c + 1) * hc, :] = jnp.dot(
            ctx, wo_ref[...], preferred_element_type=jnp.float32
        ).astype(jnp.bfloat16)

    def issue(stage, c, src_ref, dst_ref):
        ops = []
        for s in (0, 1):
            rows = pl.ds(c * hc, hc)
            cols = pl.ds(s * cw, cw)
            rdma = pltpu.make_async_remote_copy(
                src_ref=src_ref.at[rows, cols],
                dst_ref=dst_ref.at[rows, cols],
                send_sem=send_sems.at[stage * 4 + c * 2 + s],
                recv_sem=recv_sems.at[stage * 4 + c * 2 + s],
                device_id=(partner[s][stage],),
                device_id_type=pl.DeviceIdType.MESH,
            )
            rdma.start()
            ops.append(rdma)
        return ops

    rows_c = [pl.ds(0, hc), pl.ds(hc, hc)]

    compute_chunk(0)
    s1_c0 = issue(0, 0, p_s, r1)
    compute_chunk(1)
    s1_c1 = issue(0, 1, p_s, r1)

    for op in s1_c0:
        op.wait()
    a_s[rows_c[0], :] = (
        p_s[rows_c[0], :].astype(jnp.float32)
        + r1[rows_c[0], :].astype(jnp.float32)
    ).astype(jnp.bfloat16)
    s2_c0 = issue(1, 0, a_s, r2)

    for op in s1_c1:
        op.wait()
    a_s[rows_c[1], :] = (
        p_s[rows_c[1], :].astype(jnp.float32)
        + r1[rows_c[1], :].astype(jnp.float32)
    ).astype(jnp.bfloat16)
    s2_c1 = issue(1, 1, a_s, r2)

    for op in s2_c0:
        op.wait()
    out_ref[rows_c[0], :] = (
        a_s[rows_c[0], :].astype(jnp.float32)
        + r2[rows_c[0], :].astype(jnp.float32)
    ).astype(jnp.bfloat16)
    for op in s2_c1:
        op.wait()
    out_ref[rows_c[1], :] = (
        a_s[rows_c[1], :].astype(jnp.float32)
        + r2[rows_c[1], :].astype(jnp.float32)
    ).astype(jnp.bfloat16)

    @functools.partial(pl.run_scoped, sem=pltpu.SemaphoreType.REGULAR)
    def _(sem):
        for nbr in [p1, p2]:
            pl.semaphore_signal(
                sem, inc=1,
                device_id=(nbr,), device_id_type=pl.DeviceIdType.MESH,
            )
        pl.semaphore_wait(sem, 2)


def kernel(x, Wq, K_ext, V_ext, Wo):
    i = lax.axis_index("i")
    sq = x.shape[1]
    d = Wo.shape[1]
    hq_local = Wq.shape[1] // DH
    bf = jnp.bfloat16

    xb = x[0].astype(bf)
    k = lax.dynamic_slice_in_dim(
        K_ext[0], i * hq_local, hq_local, axis=1
    ).astype(bf)
    v = lax.dynamic_slice_in_dim(
        V_ext[0], i * hq_local, hq_local, axis=1
    ).astype(bf)

    out = pl.pallas_call(
        _fused_body,
        out_shape=jax.ShapeDtypeStruct((sq, d), jnp.bfloat16),
        in_specs=[pl.BlockSpec(memory_space=pltpu.VMEM)] * 5,
        out_specs=pl.BlockSpec(memory_space=pltpu.VMEM),
        scratch_shapes=[
            pltpu.VMEM((sq, d), bf),
            pltpu.VMEM((sq, d), bf),
            pltpu.VMEM((sq, d), bf),
            pltpu.VMEM((sq, d), bf),
            pltpu.SemaphoreType.DMA((8,)),
            pltpu.SemaphoreType.DMA((8,)),
        ],
        compiler_params=pltpu.CompilerParams(collective_id=0),
    )(xb, Wq.astype(bf), k, v, Wo.astype(bf))
    return out.reshape(1, sq, d)


# device time: 54308 ns/iter; 1.2560x vs baseline; 1.2560x over previous
import functools

import jax
import jax.numpy as jnp
from jax import lax
from jax.experimental import pallas as pl
from jax.experimental.pallas import tpu as pltpu

N_DEV = 4
SCALE = 0.08838834764831843
DH = 128
LOCAL_WINDOW = 128
GLOBAL_K = 32


def _fused_body(
    x_ref, wq_ref, k_ref, v_ref, wo_ref, out_ref,
    p_s, r1, a_s, r2, send_sems, recv_sems,
):
    my = lax.axis_index("i")
    p1 = my ^ 1
    p2 = 3 - my
    partner = [[p1, p2], [p2, p1]]

    sq, d = out_ref.shape
    hc = sq // 2
    cw = d // 2
    hq_local = k_ref.shape[0]

    barrier_sem = pltpu.get_barrier_semaphore()
    for nbr in [p1, p2]:
        pl.semaphore_signal(
            barrier_sem, inc=1,
            device_id=(nbr,), device_id_type=pl.DeviceIdType.MESH,
        )
    pl.semaphore_wait(barrier_sem, 2)

    qm = jnp.dot(
        x_ref[...], wq_ref[...], preferred_element_type=jnp.float32
    ).astype(jnp.bfloat16)

    def bias_band(row0, nrows, col_pieces):
        parts = []
        for c0, w in col_pieces:
            qi = lax.broadcasted_iota(jnp.int32, (nrows, w), 0) + row0
            ki = lax.broadcasted_iota(jnp.int32, (nrows, w), 1) + c0
            m = (
                (jnp.abs(qi - ki) <= LOCAL_WINDOW)
                | (ki < GLOBAL_K)
                | (qi < GLOBAL_K)
            )
            parts.append(jnp.where(m, jnp.float32(0.0), jnp.float32(-1e9)))
        return jnp.concatenate(parts, axis=1) if len(parts) > 1 else parts[0]

    chunk_bands = [
        [(0, 128, [(0, sq)]), (128, 384, [(0, 640)])],
        [(512, 512, [(0, 128), (384, 640)])],
    ]
    chunk_biases = [
        [bias_band(r0, nr, cp) for r0, nr, cp in bands]
        for bands in chunk_bands
    ]

    def compute_chunk(c):
        head_parts = []
        for hh in range(hq_local):
            band_rows = []
            for (r0, nr, cp), bias in zip(chunk_bands[c], chunk_biases[c]):
                qh = qm[r0:r0 + nr, hh * DH:(hh + 1) * DH]
                if len(cp) == 1:
                    c0, w = cp[0]
                    ksub = k_ref[hh, c0:c0 + w, :]
                    vsub = v_ref[hh, c0:c0 + w, :]
                else:
                    ksub = jnp.concatenate(
                        [k_ref[hh, c0:c0 + w, :] for c0, w in cp], axis=0
                    )
                    vsub = jnp.concatenate(
                        [v_ref[hh, c0:c0 + w, :] for c0, w in cp], axis=0
                    )
                s = lax.dot_general(
                    qh, ksub, (((1,), (1,)), ((), ())),
                    preferred_element_type=jnp.float32,
                ) * SCALE + bias
                e = jnp.exp(s)
                den = jnp.sum(e, axis=1, keepdims=True)
                w = (e / den).astype(jnp.bfloat16)
                band_rows.append(
                    jnp.dot(
                        w, vsub, preferred_element_type=jnp.float32
                    ).astype(jnp.bfloat16)
                )
            head_parts.append(
                band_rows[0] if len(band_rows) == 1
                else jnp.concatenate(band_rows, axis=0)
            )
        ctx = jnp.concatenate(head_parts, axis=1)
        p_s[c * hc:(c + 1) * hc, :] = jnp.dot(
            ctx, wo_ref[...], preferred_element_type=jnp.float32
        ).astype(jnp.bfloat16)

    def issue(stage, c, src_ref, dst_ref):
        ops = []
        for s in (0, 1):
            rows = pl.ds(c * hc, hc)
            cols = pl.ds(s * cw, cw)
            rdma = pltpu.make_async_remote_copy(
                src_ref=src_ref.at[rows, cols],
                dst_ref=dst_ref.at[rows, cols],
                send_sem=send_sems.at[stage * 4 + c * 2 + s],
                recv_sem=recv_sems.at[stage * 4 + c * 2 + s],
                device_id=(partner[s][stage],),
                device_id_type=pl.DeviceIdType.MESH,
            )
            rdma.start()
            ops.append(rdma)
        return ops

    rows_c = [pl.ds(0, hc), pl.ds(hc, hc)]

    compute_chunk(0)
    s1_c0 = issue(0, 0, p_s, r1)
    compute_chunk(1)
    s1_c1 = issue(0, 1, p_s, r1)

    for op in s1_c0:
        op.wait()
    a_s[rows_c[0], :] = (
        p_s[rows_c[0], :].astype(jnp.float32)
        + r1[rows_c[0], :].astype(jnp.float32)
    ).astype(jnp.bfloat16)
    s2_c0 = issue(1, 0, a_s, r2)

    for op in s1_c1:
        op.wait()
    a_s[rows_c[1], :] = (
        p_s[rows_c[1], :].astype(jnp.float32)
        + r1[rows_c[1], :].astype(jnp.float32)
    ).astype(jnp.bfloat16)
    s2_c1 = issue(1, 1, a_s, r2)

    for op in s2_c0:
        op.wait()
    out_ref[rows_c[0], :] = (
        a_s[rows_c[0], :].astype(jnp.float32)
        + r2[rows_c[0], :].astype(jnp.float32)
    ).astype(jnp.bfloat16)
    for op in s2_c1:
        op.wait()
    out_ref[rows_c[1], :] = (
        a_s[rows_c[1], :].astype(jnp.float32)
        + r2[rows_c[1], :].astype(jnp.float32)
    ).astype(jnp.bfloat16)

    @functools.partial(pl.run_scoped, sem=pltpu.SemaphoreType.REGULAR)
    def _(sem):
        for nbr in [p1, p2]:
            pl.semaphore_signal(
                sem, inc=1,
                device_id=(nbr,), device_id_type=pl.DeviceIdType.MESH,
            )
        pl.semaphore_wait(sem, 2)


def kernel(x, Wq, K_ext, V_ext, Wo):
    i = lax.axis_index("i")
    sq = x.shape[1]
    d = Wo.shape[1]
    hq_local = Wq.shape[1] // DH
    bf = jnp.bfloat16

    xb = x[0].astype(bf)
    k = jnp.swapaxes(
        lax.dynamic_slice_in_dim(K_ext[0], i * hq_local, hq_local, axis=1),
        0, 1,
    ).astype(bf)
    v = jnp.swapaxes(
        lax.dynamic_slice_in_dim(V_ext[0], i * hq_local, hq_local, axis=1),
        0, 1,
    ).astype(bf)

    out = pl.pallas_call(
        _fused_body,
        out_shape=jax.ShapeDtypeStruct((sq, d), jnp.bfloat16),
        in_specs=[pl.BlockSpec(memory_space=pltpu.VMEM)] * 5,
        out_specs=pl.BlockSpec(memory_space=pltpu.VMEM),
        scratch_shapes=[
            pltpu.VMEM((sq, d), bf),
            pltpu.VMEM((sq, d), bf),
            pltpu.VMEM((sq, d), bf),
            pltpu.VMEM((sq, d), bf),
            pltpu.SemaphoreType.DMA((8,)),
            pltpu.SemaphoreType.DMA((8,)),
        ],
        compiler_params=pltpu.CompilerParams(collective_id=0),
    )(xb, Wq.astype(bf), k, v, Wo.astype(bf))
    return out.reshape(1, sq, d)


# device time: 52776 ns/iter; 1.2925x vs baseline; 1.0290x over previous
import functools

import jax
import jax.numpy as jnp
from jax import lax
from jax.experimental import pallas as pl
from jax.experimental.pallas import tpu as pltpu

N_DEV = 4
SCALE = 0.08838834764831843
DH = 128
LOCAL_WINDOW = 128
GLOBAL_K = 32


def _fused_body(
    x_ref, wq_ref, k_ref, v_ref, wo_ref, out_ref,
    p_s, r1, a_s, r2, send_sems, recv_sems,
):
    my = lax.axis_index("i")
    p1 = my ^ 1
    p2 = 3 - my
    partner = [[p1, p2], [p2, p1]]

    sq, d = out_ref.shape
    hc = sq // 4
    cw = d // 2
    hq_local = k_ref.shape[0]

    barrier_sem = pltpu.get_barrier_semaphore()
    for nbr in [p1, p2]:
        pl.semaphore_signal(
            barrier_sem, inc=1,
            device_id=(nbr,), device_id_type=pl.DeviceIdType.MESH,
        )
    pl.semaphore_wait(barrier_sem, 2)

    qm = jnp.dot(
        x_ref[...], wq_ref[...], preferred_element_type=jnp.float32
    ).astype(jnp.bfloat16)

    def bias_band(row0, nrows, col_pieces):
        parts = []
        for c0, w in col_pieces:
            qi = lax.broadcasted_iota(jnp.int32, (nrows, w), 0) + row0
            ki = lax.broadcasted_iota(jnp.int32, (nrows, w), 1) + c0
            m = (
                (jnp.abs(qi - ki) <= LOCAL_WINDOW)
                | (ki < GLOBAL_K)
                | (qi < GLOBAL_K)
            )
            parts.append(jnp.where(m, jnp.float32(0.0), jnp.float32(-1e9)))
        return jnp.concatenate(parts, axis=1) if len(parts) > 1 else parts[0]

    chunk_bands = [
        [(0, 128, [(0, sq)]), (128, 128, [(0, 384)])],
        [(256, 256, [(0, 640)])],
        [(512, 256, [(0, 128), (384, 512)])],
        [(768, 256, [(0, 128), (640, 384)])],
    ]
    chunk_biases = [
        [bias_band(r0, nr, cp) for r0, nr, cp in bands]
        for bands in chunk_bands
    ]

    def compute_chunk(c):
        head_parts = []
        for hh in range(hq_local):
            band_rows = []
            for (r0, nr, cp), bias in zip(chunk_bands[c], chunk_biases[c]):
                qh = qm[r0:r0 + nr, hh * DH:(hh + 1) * DH]
                if len(cp) == 1:
                    c0, w = cp[0]
                    ksub = k_ref[hh, c0:c0 + w, :]
                    vsub = v_ref[hh, c0:c0 + w, :]
                else:
                    ksub = jnp.concatenate(
                        [k_ref[hh, c0:c0 + w, :] for c0, w in cp], axis=0
                    )
                    vsub = jnp.concatenate(
                        [v_ref[hh, c0:c0 + w, :] for c0, w in cp], axis=0
                    )
                s = lax.dot_general(
                    qh, ksub, (((1,), (1,)), ((), ())),
                    preferred_element_type=jnp.float32,
                ) * SCALE + bias
                e = jnp.exp(s)
                den = jnp.sum(e, axis=1, keepdims=True)
                w = (e / den).astype(jnp.bfloat16)
                band_rows.append(
                    jnp.dot(
                        w, vsub, preferred_element_type=jnp.float32
                    ).astype(jnp.bfloat16)
                )
            head_parts.append(
                band_rows[0] if len(band_rows) == 1
                else jnp.concatenate(band_rows, axis=0)
            )
        ctx = jnp.concatenate(head_parts, axis=1)
        p_s[c * hc:(c + 1) * hc, :] = jnp.dot(
            ctx, wo_ref[...], preferred_element_type=jnp.float32
        ).astype(jnp.bfloat16)

    def issue(stage, c, src_ref, dst_ref):
        ops = []
        for s in (0, 1):
            rows = pl.ds(c * hc, hc)
            cols = pl.ds(s * cw, cw)
            rdma = pltpu.make_async_remote_copy(
                src_ref=src_ref.at[rows, cols],
                dst_ref=dst_ref.at[rows, cols],
                send_sem=send_sems.at[stage * 8 + c * 2 + s],
                recv_sem=recv_sems.at[stage * 8 + c * 2 + s],
                device_id=(partner[s][stage],),
                device_id_type=pl.DeviceIdType.MESH,
            )
            rdma.start()
            ops.append(rdma)
        return ops

    rows_c = [pl.ds(c * hc, hc) for c in range(4)]

    def pair_sum(c):
        a_s[rows_c[c], :] = (
            p_s[rows_c[c], :].astype(jnp.float32)
            + r1[rows_c[c], :].astype(jnp.float32)
        ).astype(jnp.bfloat16)

    def store_out(c):
        out_ref[rows_c[c], :] = (
            a_s[rows_c[c], :].astype(jnp.float32)
            + r2[rows_c[c], :].astype(jnp.float32)
        ).astype(jnp.bfloat16)

    def wait_all(ops):
        for op in ops:
            op.wait()

    s1 = [None] * 4
    s2 = [None] * 4
    compute_chunk(0)
    s1[0] = issue(0, 0, p_s, r1)
    compute_chunk(1)
    s1[1] = issue(0, 1, p_s, r1)
    wait_all(s1[0])
    pair_sum(0)
    s2[0] = issue(1, 0, a_s, r2)
    compute_chunk(2)
    s1[2] = issue(0, 2, p_s, r1)
    wait_all(s1[1])
    pair_sum(1)
    s2[1] = issue(1, 1, a_s, r2)
    compute_chunk(3)
    s1[3] = issue(0, 3, p_s, r1)
    wait_all(s1[2])
    pair_sum(2)
    s2[2] = issue(1, 2, a_s, r2)
    wait_all(s2[0])
    store_out(0)
    wait_all(s1[3])
    pair_sum(3)
    s2[3] = issue(1, 3, a_s, r2)
    wait_all(s2[1])
    store_out(1)
    wait_all(s2[2])
    store_out(2)
    wait_all(s2[3])
    store_out(3)

    @functools.partial(pl.run_scoped, sem=pltpu.SemaphoreType.REGULAR)
    def _(sem):
        for nbr in [p1, p2]:
            pl.semaphore_signal(
                sem, inc=1,
                device_id=(nbr,), device_id_type=pl.DeviceIdType.MESH,
            )
        pl.semaphore_wait(sem, 2)


def kernel(x, Wq, K_ext, V_ext, Wo):
    i = lax.axis_index("i")
    sq = x.shape[1]
    d = Wo.shape[1]
    hq_local = Wq.shape[1] // DH
    bf = jnp.bfloat16

    xb = x[0].astype(bf)
    k = jnp.swapaxes(
        lax.dynamic_slice_in_dim(K_ext[0], i * hq_local, hq_local, axis=1),
        0, 1,
    ).astype(bf)
    v = jnp.swapaxes(
        lax.dynamic_slice_in_dim(V_ext[0], i * hq_local, hq_local, axis=1),
        0, 1,
    ).astype(bf)

    out = pl.pallas_call(
        _fused_body,
        out_shape=jax.ShapeDtypeStruct((sq, d), jnp.bfloat16),
        in_specs=[pl.BlockSpec(memory_space=pltpu.VMEM)] * 5,
        out_specs=pl.BlockSpec(memory_space=pltpu.VMEM),
        scratch_shapes=[
            pltpu.VMEM((sq, d), bf),
            pltpu.VMEM((sq, d), bf),
            pltpu.VMEM((sq, d), bf),
            pltpu.VMEM((sq, d), bf),
            pltpu.SemaphoreType.DMA((16,)),
            pltpu.SemaphoreType.DMA((16,)),
        ],
        compiler_params=pltpu.CompilerParams(collective_id=0),
    )(xb, Wq.astype(bf), k, v, Wo.astype(bf))
    return out.reshape(1, sq, d)


# device time: 49709 ns/iter; 1.3722x vs baseline; 1.0617x over previous
import functools

import jax
import jax.numpy as jnp
from jax import lax
from jax.experimental import pallas as pl
from jax.experimental.pallas import tpu as pltpu

N_DEV = 4
SCALE = 0.08838834764831843
DH = 128
LOCAL_WINDOW = 128
GLOBAL_K = 32


def _fused_body(
    x_ref, wq_ref, k_ref, v_ref, wo_ref, out_ref,
    p_s, r1, a_s, r2, send_sems, recv_sems,
):
    my = lax.axis_index("i")
    p1 = my ^ 1
    p2 = 3 - my
    partner = [[p1, p2], [p2, p1]]

    sq, d = out_ref.shape
    hc = sq // 4
    cw = d // 2
    hq_local = k_ref.shape[0]

    barrier_sem = pltpu.get_barrier_semaphore()
    for nbr in [p1, p2]:
        pl.semaphore_signal(
            barrier_sem, inc=1,
            device_id=(nbr,), device_id_type=pl.DeviceIdType.MESH,
        )
    pl.semaphore_wait(barrier_sem, 2)

    qm = jnp.dot(
        x_ref[...].astype(jnp.bfloat16), wq_ref[...].astype(jnp.bfloat16),
        preferred_element_type=jnp.float32,
    ).astype(jnp.bfloat16)
    wo_b = wo_ref[...].astype(jnp.bfloat16)

    def bias_band(row0, nrows, col_pieces):
        parts = []
        for c0, w in col_pieces:
            qi = lax.broadcasted_iota(jnp.int32, (nrows, w), 0) + row0
            ki = lax.broadcasted_iota(jnp.int32, (nrows, w), 1) + c0
            m = (
                (jnp.abs(qi - ki) <= LOCAL_WINDOW)
                | (ki < GLOBAL_K)
                | (qi < GLOBAL_K)
            )
            parts.append(jnp.where(m, jnp.float32(0.0), jnp.float32(-1e9)))
        return jnp.concatenate(parts, axis=1) if len(parts) > 1 else parts[0]

    chunk_bands = [
        [(0, 128, [(0, sq)]), (128, 128, [(0, 384)])],
        [(256, 256, [(0, 640)])],
        [(512, 256, [(0, 128), (384, 512)])],
        [(768, 256, [(0, 128), (640, 384)])],
    ]
    chunk_biases = [
        [bias_band(r0, nr, cp) for r0, nr, cp in bands]
        for bands in chunk_bands
    ]

    def compute_chunk(c):
        head_parts = []
        for hh in range(hq_local):
            band_rows = []
            for (r0, nr, cp), bias in zip(chunk_bands[c], chunk_biases[c]):
                qh = qm[r0:r0 + nr, hh * DH:(hh + 1) * DH]
                if len(cp) == 1:
                    c0, w = cp[0]
                    ksub = k_ref[hh, c0:c0 + w, :]
                    vsub = v_ref[hh, c0:c0 + w, :]
                else:
                    ksub = jnp.concatenate(
                        [k_ref[hh, c0:c0 + w, :] for c0, w in cp], axis=0
                    )
                    vsub = jnp.concatenate(
                        [v_ref[hh, c0:c0 + w, :] for c0, w in cp], axis=0
                    )
                s = lax.dot_general(
                    qh, ksub, (((1,), (1,)), ((), ())),
                    preferred_element_type=jnp.float32,
                ) * SCALE + bias
                e = jnp.exp(s)
                den = jnp.sum(e, axis=1, keepdims=True)
                w = (e / den).astype(jnp.bfloat16)
                band_rows.append(
                    jnp.dot(
                        w, vsub, preferred_element_type=jnp.float32
                    ).astype(jnp.bfloat16)
                )
            head_parts.append(
                band_rows[0] if len(band_rows) == 1
                else jnp.concatenate(band_rows, axis=0)
            )
        ctx = jnp.concatenate(head_parts, axis=1)
        p_s[c * hc:(c + 1) * hc, :] = jnp.dot(
            ctx, wo_b, preferred_element_type=jnp.float32
        ).astype(jnp.bfloat16)

    def issue(stage, c, src_ref, dst_ref):
        ops = []
        for s in (0, 1):
            rows = pl.ds(c * hc, hc)
            cols = pl.ds(s * cw, cw)
            rdma = pltpu.make_async_remote_copy(
                src_ref=src_ref.at[rows, cols],
                dst_ref=dst_ref.at[rows, cols],
                send_sem=send_sems.at[stage * 8 + c * 2 + s],
                recv_sem=recv_sems.at[stage * 8 + c * 2 + s],
                device_id=(partner[s][stage],),
                device_id_type=pl.DeviceIdType.MESH,
            )
            rdma.start()
            ops.append(rdma)
        return ops

    rows_c = [pl.ds(c * hc, hc) for c in range(4)]

    def pair_sum(c):
        a_s[rows_c[c], :] = (
            p_s[rows_c[c], :].astype(jnp.float32)
            + r1[rows_c[c], :].astype(jnp.float32)
        ).astype(jnp.bfloat16)

    def store_out(c):
        out_ref[rows_c[c], :] = (
            a_s[rows_c[c], :].astype(jnp.float32)
            + r2[rows_c[c], :].astype(jnp.float32)
        ).astype(jnp.bfloat16)

    def wait_all(ops):
        for op in ops:
            op.wait()

    s1 = [None] * 4
    s2 = [None] * 4
    compute_chunk(0)
    s1[0] = issue(0, 0, p_s, r1)
    compute_chunk(1)
    s1[1] = issue(0, 1, p_s, r1)
    wait_all(s1[0])
    pair_sum(0)
    s2[0] = issue(1, 0, a_s, r2)
    compute_chunk(2)
    s1[2] = issue(0, 2, p_s, r1)
    wait_all(s1[1])
    pair_sum(1)
    s2[1] = issue(1, 1, a_s, r2)
    compute_chunk(3)
    s1[3] = issue(0, 3, p_s, r1)
    wait_all(s1[2])
    pair_sum(2)
    s2[2] = issue(1, 2, a_s, r2)
    wait_all(s2[0])
    store_out(0)
    wait_all(s1[3])
    pair_sum(3)
    s2[3] = issue(1, 3, a_s, r2)
    wait_all(s2[1])
    store_out(1)
    wait_all(s2[2])
    store_out(2)
    wait_all(s2[3])
    store_out(3)

    @functools.partial(pl.run_scoped, sem=pltpu.SemaphoreType.REGULAR)
    def _(sem):
        for nbr in [p1, p2]:
            pl.semaphore_signal(
                sem, inc=1,
                device_id=(nbr,), device_id_type=pl.DeviceIdType.MESH,
            )
        pl.semaphore_wait(sem, 2)


def kernel(x, Wq, K_ext, V_ext, Wo):
    i = lax.axis_index("i")
    sq = x.shape[1]
    d = Wo.shape[1]
    hq_local = Wq.shape[1] // DH
    bf = jnp.bfloat16

    k = jnp.swapaxes(
        lax.dynamic_slice_in_dim(K_ext[0], i * hq_local, hq_local, axis=1),
        0, 1,
    ).astype(bf)
    v = jnp.swapaxes(
        lax.dynamic_slice_in_dim(V_ext[0], i * hq_local, hq_local, axis=1),
        0, 1,
    ).astype(bf)

    out = pl.pallas_call(
        _fused_body,
        out_shape=jax.ShapeDtypeStruct((sq, d), jnp.bfloat16),
        in_specs=[pl.BlockSpec(memory_space=pltpu.VMEM)] * 5,
        out_specs=pl.BlockSpec(memory_space=pltpu.VMEM),
        scratch_shapes=[
            pltpu.VMEM((sq, d), bf),
            pltpu.VMEM((sq, d), bf),
            pltpu.VMEM((sq, d), bf),
            pltpu.VMEM((sq, d), bf),
            pltpu.SemaphoreType.DMA((16,)),
            pltpu.SemaphoreType.DMA((16,)),
        ],
        compiler_params=pltpu.CompilerParams(collective_id=0),
    )(x[0], Wq, k, v, Wo)
    return out.reshape(1, sq, d)


# device time: 46193 ns/iter; 1.4767x vs baseline; 1.0761x over previous
import functools

import jax
import jax.numpy as jnp
from jax import lax
from jax.experimental import pallas as pl
from jax.experimental.pallas import tpu as pltpu

N_DEV = 4
SCALE = 0.08838834764831843
DH = 128
LOCAL_WINDOW = 128
GLOBAL_K = 32


def _fused_body(
    x_ref, wq_ref, k_ref, v_ref, wo_ref, out_ref,
    p_s, r1, a_s, r2, send_sems, recv_sems,
):
    my = lax.axis_index("i")
    p1 = my ^ 1
    p2 = 3 - my
    partner = [[p1, p2], [p2, p1]]

    sq, d = out_ref.shape
    hc = sq // 8
    cw = d // 2
    hq_local = k_ref.shape[0]

    barrier_sem = pltpu.get_barrier_semaphore()
    for nbr in [p1, p2]:
        pl.semaphore_signal(
            barrier_sem, inc=1,
            device_id=(nbr,), device_id_type=pl.DeviceIdType.MESH,
        )
    pl.semaphore_wait(barrier_sem, 2)

    qm = (jnp.dot(
        x_ref[...].astype(jnp.bfloat16), wq_ref[...].astype(jnp.bfloat16),
        preferred_element_type=jnp.float32,
    ) * SCALE).astype(jnp.bfloat16)
    wo_b = wo_ref[...].astype(jnp.bfloat16)

    def bias_band(row0, nrows, col_pieces):
        parts = []
        for c0, w in col_pieces:
            qi = lax.broadcasted_iota(jnp.int32, (nrows, w), 0) + row0
            ki = lax.broadcasted_iota(jnp.int32, (nrows, w), 1) + c0
            m = (
                (jnp.abs(qi - ki) <= LOCAL_WINDOW)
                | (ki < GLOBAL_K)
                | (qi < GLOBAL_K)
            )
            parts.append(jnp.where(m, jnp.float32(0.0), jnp.float32(-1e9)))
        return jnp.concatenate(parts, axis=1) if len(parts) > 1 else parts[0]

    chunk_bands = [
        [(0, 128, [(0, sq)])],
        [(128, 128, [(0, 384)])],
        [(256, 128, [(0, 512)])],
        [(384, 128, [(0, 640)])],
        [(512, 128, [(0, 128), (384, 384)])],
        [(640, 128, [(0, 128), (512, 384)])],
        [(768, 128, [(0, 128), (640, 384)])],
        [(896, 128, [(0, 128), (768, 256)])],
    ]
    chunk_biases = [
        [bias_band(r0, nr, cp) for r0, nr, cp in bands]
        for bands in chunk_bands
    ]

    def compute_chunk(c):
        head_parts = []
        for hh in range(hq_local):
            band_rows = []
            for (r0, nr, cp), bias in zip(chunk_bands[c], chunk_biases[c]):
                qh = qm[r0:r0 + nr, hh * DH:(hh + 1) * DH]
                if len(cp) == 1:
                    c0, w = cp[0]
                    ksub = k_ref[hh, c0:c0 + w, :]
                    vsub = v_ref[hh, c0:c0 + w, :]
                else:
                    ksub = jnp.concatenate(
                        [k_ref[hh, c0:c0 + w, :] for c0, w in cp], axis=0
                    )
                    vsub = jnp.concatenate(
                        [v_ref[hh, c0:c0 + w, :] for c0, w in cp], axis=0
                    )
                s = lax.dot_general(
                    qh, ksub, (((1,), (1,)), ((), ())),
                    preferred_element_type=jnp.float32,
                ) + bias
                e = jnp.exp(s)
                den = jnp.sum(e, axis=1, keepdims=True)
                band_rows.append(
                    (
                        jnp.dot(
                            e.astype(jnp.bfloat16), vsub,
                            preferred_element_type=jnp.float32,
                        ) / den
                    ).astype(jnp.bfloat16)
                )
            head_parts.append(
                band_rows[0] if len(band_rows) == 1
                else jnp.concatenate(band_rows, axis=0)
            )
        ctx = jnp.concatenate(head_parts, axis=1)
        p_s[c * hc:(c + 1) * hc, :] = jnp.dot(
            ctx, wo_b, preferred_element_type=jnp.float32
        ).astype(jnp.bfloat16)

    def issue(stage, c, src_ref, dst_ref):
        ops = []
        for s in (0, 1):
            rows = pl.ds(c * hc, hc)
            cols = pl.ds(s * cw, cw)
            rdma = pltpu.make_async_remote_copy(
                src_ref=src_ref.at[rows, cols],
                dst_ref=dst_ref.at[rows, cols],
                send_sem=send_sems.at[stage * 16 + c * 2 + s],
                recv_sem=recv_sems.at[stage * 16 + c * 2 + s],
                device_id=(partner[s][stage],),
                device_id_type=pl.DeviceIdType.MESH,
            )
            rdma.start()
            ops.append(rdma)
        return ops

    rows_c = [pl.ds(c * hc, hc) for c in range(8)]

    def pair_sum(c):
        a_s[rows_c[c], :] = (
            p_s[rows_c[c], :].astype(jnp.float32)
            + r1[rows_c[c], :].astype(jnp.float32)
        ).astype(jnp.bfloat16)

    def store_out(c):
        out_ref[rows_c[c], :] = (
            a_s[rows_c[c], :].astype(jnp.float32)
            + r2[rows_c[c], :].astype(jnp.float32)
        ).astype(jnp.bfloat16)

    def wait_all(ops):
        for op in ops:
            op.wait()

    NC = 8
    s1 = [None] * NC
    s2 = [None] * NC
    for c in range(NC):
        compute_chunk(c)
        s1[c] = issue(0, c, p_s, r1)
        if c >= 1:
            wait_all(s1[c - 1])
            pair_sum(c - 1)
            s2[c - 1] = issue(1, c - 1, a_s, r2)
        if c >= 4:
            wait_all(s2[c - 4])
            store_out(c - 4)
    wait_all(s1[NC - 1])
    pair_sum(NC - 1)
    s2[NC - 1] = issue(1, NC - 1, a_s, r2)
    for c in range(NC - 4, NC):
        wait_all(s2[c])
        store_out(c)

    @functools.partial(pl.run_scoped, sem=pltpu.SemaphoreType.REGULAR)
    def _(sem):
        for nbr in [p1, p2]:
            pl.semaphore_signal(
                sem, inc=1,
                device_id=(nbr,), device_id_type=pl.DeviceIdType.MESH,
            )
        pl.semaphore_wait(sem, 2)


def kernel(x, Wq, K_ext, V_ext, Wo):
    i = lax.axis_index("i")
    sq = x.shape[1]
    d = Wo.shape[1]
    hq_local = Wq.shape[1] // DH
    bf = jnp.bfloat16

    k = jnp.swapaxes(
        lax.dynamic_slice_in_dim(K_ext[0], i * hq_local, hq_local, axis=1),
        0, 1,
    ).astype(bf)
    v = jnp.swapaxes(
        lax.dynamic_slice_in_dim(V_ext[0], i * hq_local, hq_local, axis=1),
        0, 1,
    ).astype(bf)

    out = pl.pallas_call(
        _fused_body,
        out_shape=jax.ShapeDtypeStruct((sq, d), jnp.bfloat16),
        in_specs=[pl.BlockSpec(memory_space=pltpu.VMEM)] * 5,
        out_specs=pl.BlockSpec(memory_space=pltpu.VMEM),
        scratch_shapes=[
            pltpu.VMEM((sq, d), bf),
            pltpu.VMEM((sq, d), bf),
            pltpu.VMEM((sq, d), bf),
            pltpu.VMEM((sq, d), bf),
            pltpu.SemaphoreType.DMA((32,)),
            pltpu.SemaphoreType.DMA((32,)),
        ],
        compiler_params=pltpu.CompilerParams(collective_id=0),
    )(x[0], Wq, k, v, Wo)
    return out.reshape(1, sq, d)


# device time: 46165 ns/iter; 1.4776x vs baseline; 1.0006x over previous
import functools

import jax
import jax.numpy as jnp
from jax import lax
from jax.experimental import pallas as pl
from jax.experimental.pallas import tpu as pltpu

N_DEV = 4
SCALE = 0.08838834764831843
DH = 128
LOCAL_WINDOW = 128
GLOBAL_K = 32


def _fused_body(
    x_ref, wq_ref, k_ref, v_ref, wo_ref, out_ref,
    p_s, r1, a_s, r2, send_sems, recv_sems,
):
    my = lax.axis_index("i")
    p1 = my ^ 1
    p2 = 3 - my
    partner = [[p1, p2], [p2, p1]]

    sq, d = out_ref.shape
    hc = sq // 8
    cw = d // 2
    hq_local = k_ref.shape[0]

    qm = (jnp.dot(
        x_ref[...].astype(jnp.bfloat16), wq_ref[...].astype(jnp.bfloat16),
        preferred_element_type=jnp.float32,
    ) * SCALE).astype(jnp.bfloat16)
    wo_b = wo_ref[...].astype(jnp.bfloat16)

    def bias_band(row0, nrows, col_pieces):
        parts = []
        for c0, w in col_pieces:
            qi = lax.broadcasted_iota(jnp.int32, (nrows, w), 0) + row0
            ki = lax.broadcasted_iota(jnp.int32, (nrows, w), 1) + c0
            m = (
                (jnp.abs(qi - ki) <= LOCAL_WINDOW)
                | (ki < GLOBAL_K)
                | (qi < GLOBAL_K)
            )
            parts.append(jnp.where(m, jnp.float32(0.0), jnp.float32(-1e9)))
        return jnp.concatenate(parts, axis=1) if len(parts) > 1 else parts[0]

    chunk_bands = [
        [(0, 128, [(0, sq)])],
        [(128, 128, [(0, 384)])],
        [(256, 128, [(0, 512)])],
        [(384, 128, [(0, 640)])],
        [(512, 128, [(0, 128), (384, 384)])],
        [(640, 128, [(0, 128), (512, 384)])],
        [(768, 128, [(0, 128), (640, 384)])],
        [(896, 128, [(0, 128), (768, 256)])],
    ]
    chunk_biases = [
        [bias_band(r0, nr, cp) for r0, nr, cp in bands]
        for bands in chunk_bands
    ]

    def compute_chunk(c):
        head_parts = []
        for hh in range(hq_local):
            band_rows = []
            for (r0, nr, cp), bias in zip(chunk_bands[c], chunk_biases[c]):
                qh = qm[r0:r0 + nr, hh * DH:(hh + 1) * DH]
                if len(cp) == 1:
                    c0, w = cp[0]
                    ksub = k_ref[hh, c0:c0 + w, :]
                    vsub = v_ref[hh, c0:c0 + w, :]
                else:
                    ksub = jnp.concatenate(
                        [k_ref[hh, c0:c0 + w, :] for c0, w in cp], axis=0
                    )
                    vsub = jnp.concatenate(
                        [v_ref[hh, c0:c0 + w, :] for c0, w in cp], axis=0
                    )
                s = lax.dot_general(
                    qh, ksub, (((1,), (1,)), ((), ())),
                    preferred_element_type=jnp.float32,
                ) + bias
                e = jnp.exp(s)
                den = jnp.sum(e, axis=1, keepdims=True)
                band_rows.append(
                    (
                        jnp.dot(
                            e.astype(jnp.bfloat16), vsub,
                            preferred_element_type=jnp.float32,
                        ) / den
                    ).astype(jnp.bfloat16)
                )
            head_parts.append(
                band_rows[0] if len(band_rows) == 1
                else jnp.concatenate(band_rows, axis=0)
            )
        ctx = jnp.concatenate(head_parts, axis=1)
        p_s[c * hc:(c + 1) * hc, :] = jnp.dot(
            ctx, wo_b, preferred_element_type=jnp.float32
        ).astype(jnp.bfloat16)

    def issue1(stage, c, s, src_ref, dst_ref):
        rows = pl.ds(c * hc, hc)
        cols = pl.ds(s * cw, cw)
        rdma = pltpu.make_async_remote_copy(
            src_ref=src_ref.at[rows, cols],
            dst_ref=dst_ref.at[rows, cols],
            send_sem=send_sems.at[stage * 16 + c * 2 + s],
            recv_sem=recv_sems.at[stage * 16 + c * 2 + s],
            device_id=(partner[s][stage],),
            device_id_type=pl.DeviceIdType.MESH,
        )
        rdma.start()
        return rdma

    def issue(stage, c, src_ref, dst_ref):
        return [issue1(stage, c, s, src_ref, dst_ref) for s in (0, 1)]

    rows_c = [pl.ds(c * hc, hc) for c in range(8)]

    cols_s = [pl.ds(0, cw), pl.ds(cw, cw)]

    def pair_sum(c, s):
        a_s[rows_c[c], cols_s[s]] = (
            p_s[rows_c[c], cols_s[s]] + r1[rows_c[c], cols_s[s]]
        )

    def store_out(c, s):
        out_ref[rows_c[c], cols_s[s]] = (
            a_s[rows_c[c], cols_s[s]] + r2[rows_c[c], cols_s[s]]
        )

    def wait_all(ops):
        for op in ops:
            op.wait()

    NC = 8
    s1 = [None] * NC
    s2 = [None] * NC

    def advance_s1(c):
        s2[c] = [None, None]
        for s in (0, 1):
            s1[c][s].wait()
            pair_sum(c, s)
            s2[c][s] = issue1(1, c, s, a_s, r2)

    def drain_s2(c):
        for s in (0, 1):
            s2[c][s].wait()
            store_out(c, s)

    for c in range(NC):
        compute_chunk(c)
        if c == 0:
            barrier_sem = pltpu.get_barrier_semaphore()
            for nbr in [p1, p2]:
                pl.semaphore_signal(
                    barrier_sem, inc=1,
                    device_id=(nbr,), device_id_type=pl.DeviceIdType.MESH,
                )
            pl.semaphore_wait(barrier_sem, 2)
        s1[c] = issue(0, c, p_s, r1)
        if c >= 1:
            advance_s1(c - 1)
        if c >= 4:
            drain_s2(c - 4)
    advance_s1(NC - 1)
    for c in range(NC - 4, NC):
        drain_s2(c)

    @functools.partial(pl.run_scoped, sem=pltpu.SemaphoreType.REGULAR)
    def _(sem):
        for nbr in [p1, p2]:
            pl.semaphore_signal(
                sem, inc=1,
                device_id=(nbr,), device_id_type=pl.DeviceIdType.MESH,
            )
        pl.semaphore_wait(sem, 2)


def kernel(x, Wq, K_ext, V_ext, Wo):
    i = lax.axis_index("i")
    sq = x.shape[1]
    d = Wo.shape[1]
    hq_local = Wq.shape[1] // DH
    bf = jnp.bfloat16

    k = jnp.swapaxes(
        lax.dynamic_slice_in_dim(K_ext[0], i * hq_local, hq_local, axis=1),
        0, 1,
    ).astype(bf)
    v = jnp.swapaxes(
        lax.dynamic_slice_in_dim(V_ext[0], i * hq_local, hq_local, axis=1),
        0, 1,
    ).astype(bf)

    out = pl.pallas_call(
        _fused_body,
        out_shape=jax.ShapeDtypeStruct((sq, d), jnp.bfloat16),
        in_specs=[pl.BlockSpec(memory_space=pltpu.VMEM)] * 5,
        out_specs=pl.BlockSpec(memory_space=pltpu.VMEM),
        scratch_shapes=[
            pltpu.VMEM((sq, d), bf),
            pltpu.VMEM((sq, d), bf),
            pltpu.VMEM((sq, d), bf),
            pltpu.VMEM((sq, d), bf),
            pltpu.SemaphoreType.DMA((32,)),
            pltpu.SemaphoreType.DMA((32,)),
        ],
        compiler_params=pltpu.CompilerParams(collective_id=0),
    )(x[0], Wq, k, v, Wo)
    return out.reshape(1, sq, d)


# device time: 46046 ns/iter; 1.4814x vs baseline; 1.0026x over previous
import functools

import jax
import jax.numpy as jnp
from jax import lax
from jax.experimental import pallas as pl
from jax.experimental.pallas import tpu as pltpu

N_DEV = 4
SCALE = 0.08838834764831843
DH = 128
LOCAL_WINDOW = 128
GLOBAL_K = 32


def _fused_body(
    x_ref, wq_ref, k_ref, v_ref, wo_ref, out_ref,
    p_s, r1, a_s, r2, send_sems, recv_sems,
):
    my = lax.axis_index("i")
    p1 = my ^ 1
    p2 = 3 - my
    partner = [[p1, p2], [p2, p1]]

    sq, d = out_ref.shape
    hc = sq // 8
    cw = d // 2
    hq_local = k_ref.shape[0]

    qm = (jnp.dot(
        x_ref[...].astype(jnp.bfloat16), wq_ref[...].astype(jnp.bfloat16),
        preferred_element_type=jnp.float32,
    ) * SCALE).astype(jnp.bfloat16)
    wo_b = wo_ref[...].astype(jnp.bfloat16)

    def bias_band(row0, nrows, col_pieces):
        parts = []
        for c0, w in col_pieces:
            qi = lax.broadcasted_iota(jnp.int32, (nrows, w), 0) + row0
            ki = lax.broadcasted_iota(jnp.int32, (nrows, w), 1) + c0
            m = (
                (jnp.abs(qi - ki) <= LOCAL_WINDOW)
                | (ki < GLOBAL_K)
                | (qi < GLOBAL_K)
            )
            parts.append(jnp.where(m, jnp.float32(0.0), jnp.float32(-1e9)))
        return jnp.concatenate(parts, axis=1) if len(parts) > 1 else parts[0]

    chunk_bands = [
        [(0, 128, [(0, sq)])],
        [(128, 128, [(0, 384)])],
        [(256, 128, [(0, 512)])],
        [(384, 128, [(0, 640)])],
        [(512, 128, [(0, 128), (384, 384)])],
        [(640, 128, [(0, 128), (512, 384)])],
        [(768, 128, [(0, 128), (640, 384)])],
        [(896, 128, [(0, 128), (768, 256)])],
    ]
    chunk_biases = [
        [bias_band(r0, nr, cp) for r0, nr, cp in bands]
        for bands in chunk_bands
    ]

    def compute_chunk(c):
        head_parts = []
        for hh in range(hq_local):
            band_rows = []
            for (r0, nr, cp), bias in zip(chunk_bands[c], chunk_biases[c]):
                qh = qm[r0:r0 + nr, hh * DH:(hh + 1) * DH]
                if len(cp) == 1:
                    c0, w = cp[0]
                    ksub = k_ref[hh, c0:c0 + w, :]
                    vsub = v_ref[hh, c0:c0 + w, :]
                else:
                    ksub = jnp.concatenate(
                        [k_ref[hh, c0:c0 + w, :] for c0, w in cp], axis=0
                    )
                    vsub = jnp.concatenate(
                        [v_ref[hh, c0:c0 + w, :] for c0, w in cp], axis=0
                    )
                s = lax.dot_general(
                    qh, ksub, (((1,), (1,)), ((), ())),
                    preferred_element_type=jnp.float32,
                ) + bias
                e = jnp.exp(s)
                den = jnp.sum(e, axis=1, keepdims=True)
                band_rows.append(
                    (
                        jnp.dot(
                            e.astype(jnp.bfloat16), vsub,
                            preferred_element_type=jnp.float32,
                        ) / den
                    ).astype(jnp.bfloat16)
                )
            head_parts.append(
                band_rows[0] if len(band_rows) == 1
                else jnp.concatenate(band_rows, axis=0)
            )
        ctx = jnp.concatenate(head_parts, axis=1)
        p_s[c * hc:(c + 1) * hc, :] = jnp.dot(
            ctx, wo_b, preferred_element_type=jnp.float32
        ).astype(jnp.bfloat16)

    def issue1(stage, c, s, src_ref, dst_ref):
        rows = pl.ds(c * hc, hc)
        cols = pl.ds(s * cw, cw)
        rdma = pltpu.make_async_remote_copy(
            src_ref=src_ref.at[rows, cols],
            dst_ref=dst_ref.at[rows, cols],
            send_sem=send_sems.at[stage * 16 + c * 2 + s],
            recv_sem=recv_sems.at[stage * 16 + c * 2 + s],
            device_id=(partner[s][stage],),
            device_id_type=pl.DeviceIdType.MESH,
        )
        rdma.start()
        return rdma

    def issue(stage, c, src_ref, dst_ref):
        return [issue1(stage, c, s, src_ref, dst_ref) for s in (0, 1)]

    rows_c = [pl.ds(c * hc, hc) for c in range(8)]

    cols_s = [pl.ds(0, cw), pl.ds(cw, cw)]

    def pair_sum(c, s):
        a_s[rows_c[c], cols_s[s]] = (
            p_s[rows_c[c], cols_s[s]] + r1[rows_c[c], cols_s[s]]
        )

    def store_out(c, s):
        out_ref[rows_c[c], cols_s[s]] = (
            a_s[rows_c[c], cols_s[s]] + r2[rows_c[c], cols_s[s]]
        )

    def wait_all(ops):
        for op in ops:
            op.wait()

    NC = 8
    s1 = [None] * NC
    s2 = [None] * NC

    def advance_s1(c):
        s2[c] = [None, None]
        for s in (0, 1):
            s1[c][s].wait()
            pair_sum(c, s)
            s2[c][s] = issue1(1, c, s, a_s, r2)

    def drain_s2(c):
        for s in (0, 1):
            s2[c][s].wait()
            store_out(c, s)

    for c in range(NC):
        compute_chunk(c)
        if c == 0:
            barrier_sem = pltpu.get_barrier_semaphore()
            for nbr in [p1, p2]:
                pl.semaphore_signal(
                    barrier_sem, inc=1,
                    device_id=(nbr,), device_id_type=pl.DeviceIdType.MESH,
                )
            pl.semaphore_wait(barrier_sem, 2)
        s1[c] = issue(0, c, p_s, r1)
        if c >= 2:
            advance_s1(c - 2)
        if c >= 5:
            drain_s2(c - 5)
    advance_s1(NC - 2)
    advance_s1(NC - 1)
    for c in range(NC - 5, NC):
        drain_s2(c)

    @functools.partial(pl.run_scoped, sem=pltpu.SemaphoreType.REGULAR)
    def _(sem):
        for nbr in [p1, p2]:
            pl.semaphore_signal(
                sem, inc=1,
                device_id=(nbr,), device_id_type=pl.DeviceIdType.MESH,
            )
        pl.semaphore_wait(sem, 2)


def kernel(x, Wq, K_ext, V_ext, Wo):
    i = lax.axis_index("i")
    sq = x.shape[1]
    d = Wo.shape[1]
    hq_local = Wq.shape[1] // DH
    bf = jnp.bfloat16

    k = jnp.swapaxes(
        lax.dynamic_slice_in_dim(K_ext[0], i * hq_local, hq_local, axis=1),
        0, 1,
    ).astype(bf)
    v = jnp.swapaxes(
        lax.dynamic_slice_in_dim(V_ext[0], i * hq_local, hq_local, axis=1),
        0, 1,
    ).astype(bf)

    out = pl.pallas_call(
        _fused_body,
        out_shape=jax.ShapeDtypeStruct((sq, d), jnp.bfloat16),
        in_specs=[pl.BlockSpec(memory_space=pltpu.VMEM)] * 5,
        out_specs=pl.BlockSpec(memory_space=pltpu.VMEM),
        scratch_shapes=[
            pltpu.VMEM((sq, d), bf),
            pltpu.VMEM((sq, d), bf),
            pltpu.VMEM((sq, d), bf),
            pltpu.VMEM((sq, d), bf),
            pltpu.SemaphoreType.DMA((32,)),
            pltpu.SemaphoreType.DMA((32,)),
        ],
        compiler_params=pltpu.CompilerParams(collective_id=0),
    )(x[0], Wq, k, v, Wo)
    return out.reshape(1, sq, d)
